# baseline (speedup 1.0000x reference)
import functools

import jax
import jax.numpy as jnp
from jax import lax
from jax.experimental import pallas as pl
from jax.experimental.pallas import tpu as pltpu
from jax.experimental.pallas import tpu_sc as plsc

N_NODES = 10000
N_EDGES = 320000
D_NODE = 128
D_EDGE = 16
D_HIDDEN = 128
D_OUT = 128
N_GRAPHS = 64

NC = 1
NS = 16
CHUNK = 128
N_PAD = 10240
E_PAD = 327680
CH_PER_TILE = E_PAD // (NC * NS) // CHUNK
SUP = 16
ROWS_PER_TILE = N_PAD // NS
NBLK = N_PAD // CHUNK
EBLK = 512


def _mm_body(x_ref, w_ref, y_ref):
    y_ref[...] = jnp.dot(x_ref[...], w_ref[...],
                         preferred_element_type=jnp.float32,
                         precision=jax.lax.Precision.HIGHEST)


def _node_proj(x_p, w1a):
    return pl.pallas_call(
        _mm_body,
        grid=(NBLK,),
        in_specs=[
            pl.BlockSpec((CHUNK, D_NODE), lambda i: (i, 0)),
            pl.BlockSpec((D_NODE, D_HIDDEN), lambda i: (0, 0)),
        ],
        out_specs=pl.BlockSpec((CHUNK, D_HIDDEN), lambda i: (i, 0)),
        out_shape=jax.ShapeDtypeStruct((N_PAD, D_HIDDEN), jnp.float32),
    )(x_p, w1a)


def _edge_proj(ea_p, w1b):
    return pl.pallas_call(
        _mm_body,
        grid=(E_PAD // EBLK,),
        in_specs=[
            pl.BlockSpec((EBLK, D_EDGE), lambda i: (i, 0)),
            pl.BlockSpec((D_EDGE, D_HIDDEN), lambda i: (0, 0)),
        ],
        out_specs=pl.BlockSpec((EBLK, D_HIDDEN), lambda i: (i, 0)),
        out_shape=jax.ShapeDtypeStruct((E_PAD, D_HIDDEN), jnp.float32),
    )(ea_p, w1b)


def _sc_body(y_hbm, p_hbm, src_hbm, dst_hbm, zy_hbm, agg_out,
             src_v, dst_v, rows_v, p_v, agg_s, sem):
    c = lax.axis_index("c")
    s = lax.axis_index("s")
    wid = s * NC + c

    rbase = s * ROWS_PER_TILE
    tbase = wid * CH_PER_TILE

    pltpu.sync_copy(zy_hbm, rows_v)
    for k in range(ROWS_PER_TILE // CHUNK):
        pltpu.sync_copy(rows_v, agg_s.at[pl.ds(rbase + k * CHUNK, CHUNK)])
    plsc.subcore_barrier()

    def outer(g, carry):
        gbase = tbase + g * SUP
        pltpu.sync_copy(src_hbm.at[pl.ds(gbase, SUP)], src_v)
        pltpu.sync_copy(dst_hbm.at[pl.ds(gbase, SUP)], dst_v)

        def body(j, carry2):
            ebase = (gbase + j) * CHUNK
            pltpu.async_copy(y_hbm.at[src_v.at[j]], rows_v, sem).wait()
            pltpu.sync_copy(p_hbm.at[pl.ds(ebase, CHUNK)], p_v)
            pltpu.sync_copy(rows_v, agg_s.at[dst_v.at[j]], add=True)
            pltpu.sync_copy(p_v, agg_s.at[dst_v.at[j]], add=True)
            return carry2

        lax.fori_loop(0, SUP, body, 0)
        return carry

    lax.fori_loop(0, CH_PER_TILE // SUP, outer, 0)
    plsc.subcore_barrier()

    for k in range(ROWS_PER_TILE // CHUNK):
        row0 = rbase + k * CHUNK
        pltpu.sync_copy(agg_s.at[pl.ds(row0, CHUNK)], rows_v)
        pltpu.sync_copy(rows_v, agg_out.at[c, pl.ds(row0, CHUNK)])


def _sc_edge(y, p, src_p, dst_p, zy):
    mesh = plsc.VectorSubcoreMesh(core_axis_name="c", subcore_axis_name="s",
                                  num_cores=NC)
    fn = functools.partial(
        pl.kernel,
        mesh=mesh,
        out_type=jax.ShapeDtypeStruct((NC, N_PAD, D_HIDDEN), jnp.float32),
        scratch_types=[
            pltpu.VMEM((SUP, CHUNK), jnp.int32),
            pltpu.VMEM((SUP, CHUNK), jnp.int32),
            pltpu.VMEM((CHUNK, D_HIDDEN), jnp.float32),
            pltpu.VMEM((CHUNK, D_HIDDEN), jnp.float32),
            pltpu.VMEM_SHARED((N_PAD, D_HIDDEN), jnp.float32),
            pltpu.SemaphoreType.DMA,
        ],
    )(_sc_body)
    return fn(y, p, src_p, dst_p, zy)


def _post_body(agg_ref, b1_ref, w2_ref, b2_ref, batch_ref,
               out_ref, sums_ref, cnts_ref):
    i = pl.program_id(0)

    @pl.when(i == 0)
    def _():
        sums_ref[...] = jnp.zeros_like(sums_ref)
        cnts_ref[...] = jnp.zeros_like(cnts_ref)

    pre = sum(agg_ref[i] for i in range(NC)) + b1_ref[...]
    h = jnp.maximum(pre, 0.0)
    h2 = jnp.dot(h, w2_ref[...],
                 preferred_element_type=jnp.float32,
                 precision=jax.lax.Precision.HIGHEST) + b2_ref[...]

    gid = lax.broadcasted_iota(jnp.int32, (N_GRAPHS, CHUNK), 0)
    mask = (batch_ref[0] == gid).astype(jnp.float32)
    sums_ref[...] = sums_ref[...] + jnp.dot(
        mask, h2, preferred_element_type=jnp.float32,
        precision=jax.lax.Precision.HIGHEST)
    cnts_ref[...] = cnts_ref[...] + jnp.sum(mask, axis=1, keepdims=True)

    @pl.when(i == NBLK - 1)
    def _():
        out_ref[...] = sums_ref[...] / jnp.maximum(cnts_ref[...], 1.0)


def _post(agg2, b1r, w2, b2r, batch3):
    return pl.pallas_call(
        _post_body,
        grid=(NBLK,),
        in_specs=[
            pl.BlockSpec((NC, CHUNK, D_HIDDEN), lambda i: (0, i, 0)),
            pl.BlockSpec((1, D_HIDDEN), lambda i: (0, 0)),
            pl.BlockSpec((D_HIDDEN, D_OUT), lambda i: (0, 0)),
            pl.BlockSpec((1, D_OUT), lambda i: (0, 0)),
            pl.BlockSpec((1, 1, CHUNK), lambda i: (i, 0, 0)),
        ],
        out_specs=pl.BlockSpec((N_GRAPHS, D_OUT), lambda i: (0, 0)),
        out_shape=jax.ShapeDtypeStruct((N_GRAPHS, D_OUT), jnp.float32),
        scratch_shapes=[
            pltpu.VMEM((N_GRAPHS, D_OUT), jnp.float32),
            pltpu.VMEM((N_GRAPHS, 1), jnp.float32),
        ],
    )(agg2, b1r, w2, b2r, batch3)


@jax.jit
def kernel(x, edge_index, edge_attr, batch, W1, b1, W2, b2):
    src = edge_index[0].astype(jnp.int32)
    dst = edge_index[1].astype(jnp.int32)
    pad_e = E_PAD - N_EDGES
    src_p = jnp.concatenate(
        [src, jnp.full((pad_e,), N_NODES, jnp.int32)]).reshape(E_PAD // CHUNK, CHUNK)
    dst_p = jnp.concatenate(
        [dst, jnp.full((pad_e,), N_NODES, jnp.int32)]).reshape(E_PAD // CHUNK, CHUNK)
    ea_p = jnp.pad(edge_attr, ((0, pad_e), (0, 0)))
    x_p = jnp.pad(x, ((0, N_PAD - N_NODES), (0, 0)))
    batch3 = jnp.pad(batch.astype(jnp.int32), (0, N_PAD - N_NODES),
                     constant_values=N_GRAPHS).reshape(NBLK, 1, CHUNK)
    zy = jnp.zeros((CHUNK, D_HIDDEN), jnp.float32)
    w1a = W1[:D_NODE]
    w1b = W1[D_NODE:]
    b1r = b1.reshape(1, D_HIDDEN)
    b2r = b2.reshape(1, D_OUT)

    y = _node_proj(x_p, w1a)
    p = _edge_proj(ea_p, w1b)
    agg2 = _sc_edge(y, p, src_p, dst_p, zy)
    return _post(agg2, b1r, W2, b2r, batch3)


# speedup vs baseline: 1.2765x; 1.2765x over previous
import functools

import jax
import jax.numpy as jnp
from jax import lax
from jax.experimental import pallas as pl
from jax.experimental.pallas import tpu as pltpu
from jax.experimental.pallas import tpu_sc as plsc

N_NODES = 10000
N_EDGES = 320000
D_NODE = 128
D_EDGE = 16
D_HIDDEN = 128
D_OUT = 128
N_GRAPHS = 64

NC = 2
NS = 16
CHUNK = 128
N_PAD = 10240
E_PAD = 327680
ROWS_PER_TILE = N_PAD // NS
NBLK = N_PAD // CHUNK
EBLK = 512
CH2 = 64
NCH = E_PAD // CH2
CHT = NCH // (NC * NS)
HALF = CHT // 4
NPAIR = HALF // 2


def _mm_body(x_ref, w_ref, y_ref):
    y_ref[...] = jnp.dot(x_ref[...], w_ref[...],
                         preferred_element_type=jnp.float32,
                         precision=jax.lax.Precision.HIGHEST)


def _node_proj(x_p, w1a):
    return pl.pallas_call(
        _mm_body,
        grid=(NBLK,),
        in_specs=[
            pl.BlockSpec((CHUNK, D_NODE), lambda i: (i, 0)),
            pl.BlockSpec((D_NODE, D_HIDDEN), lambda i: (0, 0)),
        ],
        out_specs=pl.BlockSpec((CHUNK, D_HIDDEN), lambda i: (i, 0)),
        out_shape=jax.ShapeDtypeStruct((N_PAD, D_HIDDEN), jnp.float32),
    )(x_p, w1a)


def _edge_proj(ea_p, w1b):
    return pl.pallas_call(
        _mm_body,
        grid=(E_PAD // EBLK,),
        in_specs=[
            pl.BlockSpec((EBLK, D_EDGE), lambda i: (i, 0)),
            pl.BlockSpec((D_EDGE, D_HIDDEN), lambda i: (0, 0)),
        ],
        out_specs=pl.BlockSpec((EBLK, D_HIDDEN), lambda i: (i, 0)),
        out_shape=jax.ShapeDtypeStruct((E_PAD, D_HIDDEN), jnp.float32),
    )(ea_p, w1b)


def _sc_body(y_hbm, p_hbm, src_hbm, dst2_hbm, zy_hbm, agg_out,
             src_v, dst2_v, rp0, rp1, agg_s, sg0, sg1, ss0, ss1):
    c = lax.axis_index("c")
    s = lax.axis_index("s")
    wid = s * NC + c

    rbase = s * ROWS_PER_TILE
    tchunk = wid * CHT

    pltpu.sync_copy(zy_hbm, rp0)
    for k in range(ROWS_PER_TILE // CHUNK):
        pltpu.sync_copy(rp0, agg_s.at[pl.ds(rbase + k * CHUNK, CHUNK)])
    plsc.subcore_barrier()

    def half(h, carry):
        hbase = tchunk + h * HALF
        pltpu.sync_copy(src_hbm.at[pl.ds(hbase, HALF)], src_v)
        pltpu.sync_copy(dst2_hbm.at[pl.ds(hbase, HALF)], dst2_v)

        def load(row, rp, sem):
            pltpu.async_copy(y_hbm.at[src_v.at[row]], rp.at[pl.ds(0, CH2)], sem)
            pltpu.async_copy(p_hbm.at[pl.ds((hbase + row) * CH2, CH2)],
                             rp.at[pl.ds(CH2, CH2)], sem)

        def wait_load(row, rp, sem):
            pltpu.make_async_copy(y_hbm.at[src_v.at[row]],
                                  rp.at[pl.ds(0, CH2)], sem).wait()
            pltpu.make_async_copy(p_hbm.at[pl.ds(0, CH2)],
                                  rp.at[pl.ds(CH2, CH2)], sem).wait()

        def scat(row, rp, sem):
            pltpu.async_copy(rp, agg_s.at[dst2_v.at[row]], sem, add=True)

        def wait_scat(row, rp, sem):
            pltpu.make_async_copy(rp, agg_s.at[dst2_v.at[row]], sem).wait()

        load(0, rp0, sg0)
        load(1, rp1, sg1)

        def pair(g, carry2):
            r0 = 2 * g
            r1 = 2 * g + 1
            wait_load(r0, rp0, sg0)
            scat(r0, rp0, ss0)
            wait_load(r1, rp1, sg1)
            scat(r1, rp1, ss1)

            @pl.when(g < NPAIR - 1)
            def _():
                wait_scat(r0, rp0, ss0)
                load(r0 + 2, rp0, sg0)
                wait_scat(r1, rp1, ss1)
                load(r1 + 2, rp1, sg1)

            @pl.when(g == NPAIR - 1)
            def _():
                wait_scat(r0, rp0, ss0)
                wait_scat(r1, rp1, ss1)

            return carry2

        lax.fori_loop(0, NPAIR, pair, 0)
        return carry

    lax.fori_loop(0, CHT // HALF, half, 0)
    plsc.subcore_barrier()

    for k in range(ROWS_PER_TILE // CHUNK):
        row0 = rbase + k * CHUNK
        pltpu.sync_copy(agg_s.at[pl.ds(row0, CHUNK)], rp0)
        pltpu.sync_copy(rp0, agg_out.at[c, pl.ds(row0, CHUNK)])


def _sc_edge(y, p, src64, dst2, zy):
    mesh = plsc.VectorSubcoreMesh(core_axis_name="c", subcore_axis_name="s",
                                  num_cores=NC)
    fn = functools.partial(
        pl.kernel,
        mesh=mesh,
        out_type=jax.ShapeDtypeStruct((NC, N_PAD, D_HIDDEN), jnp.float32),
        scratch_types=[
            pltpu.VMEM((HALF, CH2), jnp.int32),
            pltpu.VMEM((HALF, 2 * CH2), jnp.int32),
            pltpu.VMEM((CHUNK, D_HIDDEN), jnp.float32),
            pltpu.VMEM((CHUNK, D_HIDDEN), jnp.float32),
            pltpu.VMEM_SHARED((N_PAD, D_HIDDEN), jnp.float32),
            pltpu.SemaphoreType.DMA,
            pltpu.SemaphoreType.DMA,
            pltpu.SemaphoreType.DMA,
            pltpu.SemaphoreType.DMA,
        ],
    )(_sc_body)
    return fn(y, p, src64, dst2, zy)


def _post_body(agg_ref, b1_ref, w2_ref, b2_ref, batch_ref,
               out_ref, sums_ref, cnts_ref):
    i = pl.program_id(0)

    @pl.when(i == 0)
    def _():
        sums_ref[...] = jnp.zeros_like(sums_ref)
        cnts_ref[...] = jnp.zeros_like(cnts_ref)

    pre = sum(agg_ref[i] for i in range(NC)) + b1_ref[...]
    h = jnp.maximum(pre, 0.0)
    h2 = jnp.dot(h, w2_ref[...],
                 preferred_element_type=jnp.float32,
                 precision=jax.lax.Precision.HIGHEST) + b2_ref[...]

    gid = lax.broadcasted_iota(jnp.int32, (N_GRAPHS, CHUNK), 0)
    mask = (batch_ref[0] == gid).astype(jnp.float32)
    sums_ref[...] = sums_ref[...] + jnp.dot(
        mask, h2, preferred_element_type=jnp.float32,
        precision=jax.lax.Precision.HIGHEST)
    cnts_ref[...] = cnts_ref[...] + jnp.sum(mask, axis=1, keepdims=True)

    @pl.when(i == NBLK - 1)
    def _():
        out_ref[...] = sums_ref[...] / jnp.maximum(cnts_ref[...], 1.0)


def _post(agg2, b1r, w2, b2r, batch3):
    return pl.pallas_call(
        _post_body,
        grid=(NBLK,),
        in_specs=[
            pl.BlockSpec((NC, CHUNK, D_HIDDEN), lambda i: (0, i, 0)),
            pl.BlockSpec((1, D_HIDDEN), lambda i: (0, 0)),
            pl.BlockSpec((D_HIDDEN, D_OUT), lambda i: (0, 0)),
            pl.BlockSpec((1, D_OUT), lambda i: (0, 0)),
            pl.BlockSpec((1, 1, CHUNK), lambda i: (i, 0, 0)),
        ],
        out_specs=pl.BlockSpec((N_GRAPHS, D_OUT), lambda i: (0, 0)),
        out_shape=jax.ShapeDtypeStruct((N_GRAPHS, D_OUT), jnp.float32),
        scratch_shapes=[
            pltpu.VMEM((N_GRAPHS, D_OUT), jnp.float32),
            pltpu.VMEM((N_GRAPHS, 1), jnp.float32),
        ],
    )(agg2, b1r, w2, b2r, batch3)


@jax.jit
def kernel(x, edge_index, edge_attr, batch, W1, b1, W2, b2):
    src = edge_index[0].astype(jnp.int32)
    dst = edge_index[1].astype(jnp.int32)
    pad_e = E_PAD - N_EDGES
    src64 = jnp.concatenate(
        [src, jnp.full((pad_e,), N_NODES, jnp.int32)]).reshape(NCH, CH2)
    dst64 = jnp.concatenate(
        [dst, jnp.full((pad_e,), N_NODES, jnp.int32)]).reshape(NCH, CH2)
    dst2 = jnp.concatenate([dst64, dst64], axis=1)
    ea_p = jnp.pad(edge_attr, ((0, pad_e), (0, 0)))
    x_p = jnp.pad(x, ((0, N_PAD - N_NODES), (0, 0)))
    batch3 = jnp.pad(batch.astype(jnp.int32), (0, N_PAD - N_NODES),
                     constant_values=N_GRAPHS).reshape(NBLK, 1, CHUNK)
    zy = jnp.zeros((CHUNK, D_HIDDEN), jnp.float32)
    w1a = W1[:D_NODE]
    w1b = W1[D_NODE:]
    b1r = b1.reshape(1, D_HIDDEN)
    b2r = b2.reshape(1, D_OUT)

    y = _node_proj(x_p, w1a)
    p = _edge_proj(ea_p, w1b)
    agg2 = _sc_edge(y, p, src64, dst2, zy)
    return _post(agg2, b1r, W2, b2r, batch3)


# speedup vs baseline: 2.2257x; 1.7436x over previous
import functools

import jax
import jax.numpy as jnp
from jax import lax
from jax.experimental import pallas as pl
from jax.experimental.pallas import tpu as pltpu
from jax.experimental.pallas import tpu_sc as plsc

N_NODES = 10000
N_EDGES = 320000
D_NODE = 128
D_EDGE = 16
D_HIDDEN = 128
D_OUT = 128
N_GRAPHS = 64

NC = 2
NS = 16
CHUNK = 128
N_PAD = 10240
E_PAD = 327680
ROWS_PER_TILE = N_PAD // NS
NBLK = N_PAD // CHUNK
EBLK = 4096
CH2 = 64
NCH = E_PAD // CH2
CHT0 = 240
CHT1 = 80
HALF = 40
NPAIR = HALF // 2


def _mm_body(x_ref, w_ref, y_ref):
    y_ref[...] = jnp.dot(x_ref[...], w_ref[...],
                         preferred_element_type=jnp.float32)


def _node_proj(x_p, w1a):
    return pl.pallas_call(
        _mm_body,
        grid=(NBLK,),
        in_specs=[
            pl.BlockSpec((CHUNK, D_NODE), lambda i: (i, 0)),
            pl.BlockSpec((D_NODE, D_HIDDEN), lambda i: (0, 0)),
        ],
        out_specs=pl.BlockSpec((CHUNK, D_HIDDEN), lambda i: (i, 0)),
        out_shape=jax.ShapeDtypeStruct((N_PAD, D_HIDDEN), jnp.float32),
    )(x_p, w1a)


def _ep_body(ea_ref, w_ref, p_ref):
    p_ref[...] = jnp.dot(ea_ref[...], w_ref[...],
                         preferred_element_type=jnp.float32)


def _edge_proj(ea, w1b):
    return pl.pallas_call(
        _ep_body,
        grid=(E_PAD // EBLK,),
        in_specs=[
            pl.BlockSpec((EBLK, D_EDGE),
                         lambda i: (jnp.minimum(i, N_EDGES // EBLK - 1), 0)),
            pl.BlockSpec((D_EDGE, D_HIDDEN), lambda i: (0, 0)),
        ],
        out_specs=pl.BlockSpec((EBLK, D_HIDDEN), lambda i: (i, 0)),
        out_shape=jax.ShapeDtypeStruct((E_PAD, D_HIDDEN), jnp.float32),
    )(ea, w1b)


def _sc_body(y_hbm, p_hbm, src_hbm, dst2_hbm, zy_hbm, agg_out,
             src_v, dst2_v, rp0, rp1, agg_s, sg0, sg1, ss0, ss1):
    c = lax.axis_index("c")
    s = lax.axis_index("s")
    wid = s * NC + c

    del wid
    rbase = s * ROWS_PER_TILE
    tchunk = jnp.where(c == 0, s * CHT0, NS * CHT0 + s * CHT1)
    nstage = jnp.where(c == 0, CHT0 // HALF, CHT1 // HALF)

    pltpu.sync_copy(zy_hbm, rp0)
    for k in range(ROWS_PER_TILE // CHUNK):
        pltpu.sync_copy(rp0, agg_s.at[pl.ds(rbase + k * CHUNK, CHUNK)])
    plsc.subcore_barrier()

    def half(h, carry):
        hbase = tchunk + h * HALF
        pltpu.sync_copy(src_hbm.at[pl.ds(hbase, HALF)], src_v)
        pltpu.sync_copy(dst2_hbm.at[pl.ds(hbase, HALF)], dst2_v)

        def load(row, rp, sem):
            pltpu.async_copy(y_hbm.at[src_v.at[row]], rp.at[pl.ds(0, CH2)], sem)
            pltpu.async_copy(p_hbm.at[pl.ds((hbase + row) * CH2, CH2)],
                             rp.at[pl.ds(CH2, CH2)], sem)

        def wait_load(row, rp, sem):
            pltpu.make_async_copy(y_hbm.at[src_v.at[row]],
                                  rp.at[pl.ds(0, CH2)], sem).wait()
            pltpu.make_async_copy(p_hbm.at[pl.ds(0, CH2)],
                                  rp.at[pl.ds(CH2, CH2)], sem).wait()

        def scat(row, rp, sem):
            pltpu.async_copy(rp, agg_s.at[dst2_v.at[row]], sem, add=True)

        def wait_scat(row, rp, sem):
            pltpu.make_async_copy(rp, agg_s.at[dst2_v.at[row]], sem).wait()

        load(0, rp0, sg0)
        load(1, rp1, sg1)

        def pair(g, carry2):
            r0 = 2 * g
            r1 = 2 * g + 1
            wait_load(r0, rp0, sg0)
            scat(r0, rp0, ss0)
            wait_load(r1, rp1, sg1)
            scat(r1, rp1, ss1)

            @pl.when(g < NPAIR - 1)
            def _():
                wait_scat(r0, rp0, ss0)
                load(r0 + 2, rp0, sg0)
                wait_scat(r1, rp1, ss1)
                load(r1 + 2, rp1, sg1)

            @pl.when(g == NPAIR - 1)
            def _():
                wait_scat(r0, rp0, ss0)
                wait_scat(r1, rp1, ss1)

            return carry2

        lax.fori_loop(0, NPAIR, pair, 0)
        return carry

    def half_guarded(h, carry):
        @pl.when(h < nstage)
        def _():
            half(h, 0)

        return carry

    lax.fori_loop(0, CHT0 // HALF, half_guarded, 0)
    plsc.subcore_barrier()

    for k in range(ROWS_PER_TILE // CHUNK):
        row0 = rbase + k * CHUNK
        pltpu.sync_copy(agg_s.at[pl.ds(row0, CHUNK)], rp0)
        pltpu.sync_copy(rp0, agg_out.at[c, pl.ds(row0, CHUNK)])


def _sc_edge(y, p, src64, dst2, zy):
    mesh = plsc.VectorSubcoreMesh(core_axis_name="c", subcore_axis_name="s",
                                  num_cores=NC)
    fn = functools.partial(
        pl.kernel,
        mesh=mesh,
        out_type=jax.ShapeDtypeStruct((NC, N_PAD, D_HIDDEN), jnp.float32),
        scratch_types=[
            pltpu.VMEM((HALF, CH2), jnp.int32),
            pltpu.VMEM((HALF, 2 * CH2), jnp.int32),
            pltpu.VMEM((CHUNK, D_HIDDEN), jnp.float32),
            pltpu.VMEM((CHUNK, D_HIDDEN), jnp.float32),
            pltpu.VMEM_SHARED((N_PAD, D_HIDDEN), jnp.float32),
            pltpu.SemaphoreType.DMA,
            pltpu.SemaphoreType.DMA,
            pltpu.SemaphoreType.DMA,
            pltpu.SemaphoreType.DMA,
        ],
    )(_sc_body)
    return fn(y, p, src64, dst2, zy)


def _post_body(agg_ref, b1_ref, w2_ref, b2_ref, batch_ref,
               out_ref, sums_ref, cnts_ref):
    i = pl.program_id(0)

    @pl.when(i == 0)
    def _():
        sums_ref[...] = jnp.zeros_like(sums_ref)
        cnts_ref[...] = jnp.zeros_like(cnts_ref)

    pre = sum(agg_ref[i] for i in range(NC)) + b1_ref[...]
    h = jnp.maximum(pre, 0.0)
    h2 = jnp.dot(h, w2_ref[...],
                 preferred_element_type=jnp.float32,
                 precision=jax.lax.Precision.HIGHEST) + b2_ref[...]

    gid = lax.broadcasted_iota(jnp.int32, (N_GRAPHS, CHUNK), 0)
    mask = (batch_ref[0] == gid).astype(jnp.float32)
    sums_ref[...] = sums_ref[...] + jnp.dot(
        mask, h2, preferred_element_type=jnp.float32,
        precision=jax.lax.Precision.HIGHEST)
    cnts_ref[...] = cnts_ref[...] + jnp.sum(mask, axis=1, keepdims=True)

    @pl.when(i == NBLK - 1)
    def _():
        out_ref[...] = sums_ref[...] / jnp.maximum(cnts_ref[...], 1.0)


def _post(agg2, b1r, w2, b2r, batch3):
    return pl.pallas_call(
        _post_body,
        grid=(NBLK,),
        in_specs=[
            pl.BlockSpec((NC, CHUNK, D_HIDDEN), lambda i: (0, i, 0)),
            pl.BlockSpec((1, D_HIDDEN), lambda i: (0, 0)),
            pl.BlockSpec((D_HIDDEN, D_OUT), lambda i: (0, 0)),
            pl.BlockSpec((1, D_OUT), lambda i: (0, 0)),
            pl.BlockSpec((1, 1, CHUNK), lambda i: (i, 0, 0)),
        ],
        out_specs=pl.BlockSpec((N_GRAPHS, D_OUT), lambda i: (0, 0)),
        out_shape=jax.ShapeDtypeStruct((N_GRAPHS, D_OUT), jnp.float32),
        scratch_shapes=[
            pltpu.VMEM((N_GRAPHS, D_OUT), jnp.float32),
            pltpu.VMEM((N_GRAPHS, 1), jnp.float32),
        ],
    )(agg2, b1r, w2, b2r, batch3)


@jax.jit
def kernel(x, edge_index, edge_attr, batch, W1, b1, W2, b2):
    src = edge_index[0].astype(jnp.int32)
    dst = edge_index[1].astype(jnp.int32)
    pad_e = E_PAD - N_EDGES
    src64 = jnp.concatenate(
        [src, jnp.full((pad_e,), N_NODES, jnp.int32)]).reshape(NCH, CH2)
    dst64 = jnp.concatenate(
        [dst, jnp.full((pad_e,), N_NODES, jnp.int32)]).reshape(NCH, CH2)
    dst2 = jnp.concatenate([dst64, dst64], axis=1)
    x_p = jnp.pad(x, ((0, N_PAD - N_NODES), (0, 0)))
    batch3 = jnp.pad(batch.astype(jnp.int32), (0, N_PAD - N_NODES),
                     constant_values=N_GRAPHS).reshape(NBLK, 1, CHUNK)
    zy = jnp.zeros((CHUNK, D_HIDDEN), jnp.float32)
    w1a = W1[:D_NODE]
    w1b = W1[D_NODE:]
    b1r = b1.reshape(1, D_HIDDEN)
    b2r = b2.reshape(1, D_OUT)

    y = _node_proj(x_p, w1a)
    p = _edge_proj(edge_attr, w1b)
    agg2 = _sc_edge(y, p, src64, dst2, zy)
    return _post(agg2, b1r, W2, b2r, batch3)


# speedup vs baseline: 2.3900x; 1.0738x over previous
import functools

import jax
import jax.numpy as jnp
from jax import lax
from jax.experimental import pallas as pl
from jax.experimental.pallas import tpu as pltpu
from jax.experimental.pallas import tpu_sc as plsc

N_NODES = 10000
N_EDGES = 320000
D_NODE = 128
D_EDGE = 16
D_HIDDEN = 128
D_OUT = 128
N_GRAPHS = 64

NC = 2
NS = 16
CHUNK = 128
N_PAD = 10240
E_PAD = 327680
ROWS_PER_TILE = N_PAD // NS
NBLK = N_PAD // CHUNK
EBLK = 16384
CH2 = 64
NCH = E_PAD // CH2
CHT0 = 240
CHT1 = 80
HALF = 40
NPAIR = HALF // 2


def _mm_body(x_ref, w_ref, y_ref):
    y_ref[...] = jnp.dot(x_ref[...], w_ref[...],
                         preferred_element_type=jnp.float32)


NROWBLK = 1024


def _node_proj(x_p, w1a):
    return pl.pallas_call(
        _mm_body,
        grid=(N_PAD // NROWBLK,),
        in_specs=[
            pl.BlockSpec((NROWBLK, D_NODE), lambda i: (i, 0)),
            pl.BlockSpec((D_NODE, D_HIDDEN), lambda i: (0, 0)),
        ],
        out_specs=pl.BlockSpec((NROWBLK, D_HIDDEN), lambda i: (i, 0)),
        out_shape=jax.ShapeDtypeStruct((N_PAD, D_HIDDEN), jnp.float32),
    )(x_p, w1a)


def _ep_body(ea_ref, w_ref, p_ref):
    p_ref[...] = jnp.dot(ea_ref[...], w_ref[...],
                         preferred_element_type=jnp.float32)


def _edge_proj(ea, w1b):
    return pl.pallas_call(
        _ep_body,
        grid=(E_PAD // EBLK,),
        in_specs=[
            pl.BlockSpec((EBLK, D_EDGE),
                         lambda i: (jnp.minimum(i, N_EDGES // EBLK - 1), 0)),
            pl.BlockSpec((D_EDGE, D_HIDDEN), lambda i: (0, 0)),
        ],
        out_specs=pl.BlockSpec((EBLK, D_HIDDEN), lambda i: (i, 0)),
        out_shape=jax.ShapeDtypeStruct((E_PAD, D_HIDDEN), jnp.float32),
    )(ea, w1b)


def _sc_body(y_hbm, p_hbm, src_hbm, dst2_hbm, zy_hbm, agg_out,
             src_v, dst2_v, rp0, rp1, agg_s, sg0, sg1, ss0, ss1):
    c = lax.axis_index("c")
    s = lax.axis_index("s")
    wid = s * NC + c

    del wid
    rbase = s * ROWS_PER_TILE
    tchunk = jnp.where(c == 0, s * CHT0, NS * CHT0 + s * CHT1)
    nstage = jnp.where(c == 0, CHT0 // HALF, CHT1 // HALF)

    pltpu.sync_copy(zy_hbm, rp0)
    for k in range(ROWS_PER_TILE // CHUNK):
        pltpu.sync_copy(rp0, agg_s.at[pl.ds(rbase + k * CHUNK, CHUNK)])
    plsc.subcore_barrier()

    def half(h, carry):
        hbase = tchunk + h * HALF
        pltpu.sync_copy(src_hbm.at[pl.ds(hbase, HALF)], src_v)
        pltpu.sync_copy(dst2_hbm.at[pl.ds(hbase, HALF)], dst2_v)

        def load(row, rp, sem):
            pltpu.async_copy(y_hbm.at[src_v.at[row]], rp.at[pl.ds(0, CH2)], sem)
            pltpu.async_copy(p_hbm.at[pl.ds((hbase + row) * CH2, CH2)],
                             rp.at[pl.ds(CH2, CH2)], sem)

        def wait_load(row, rp, sem):
            pltpu.make_async_copy(y_hbm.at[src_v.at[row]],
                                  rp.at[pl.ds(0, CH2)], sem).wait()
            pltpu.make_async_copy(p_hbm.at[pl.ds(0, CH2)],
                                  rp.at[pl.ds(CH2, CH2)], sem).wait()

        def scat(row, rp, sem):
            pltpu.async_copy(rp, agg_s.at[dst2_v.at[row]], sem, add=True)

        def wait_scat(row, rp, sem):
            pltpu.make_async_copy(rp, agg_s.at[dst2_v.at[row]], sem).wait()

        load(0, rp0, sg0)
        load(1, rp1, sg1)

        def pair(g, carry2):
            r0 = 2 * g
            r1 = 2 * g + 1
            wait_load(r0, rp0, sg0)
            scat(r0, rp0, ss0)
            wait_load(r1, rp1, sg1)
            scat(r1, rp1, ss1)

            @pl.when(g < NPAIR - 1)
            def _():
                wait_scat(r0, rp0, ss0)
                load(r0 + 2, rp0, sg0)
                wait_scat(r1, rp1, ss1)
                load(r1 + 2, rp1, sg1)

            @pl.when(g == NPAIR - 1)
            def _():
                wait_scat(r0, rp0, ss0)
                wait_scat(r1, rp1, ss1)

            return carry2

        lax.fori_loop(0, NPAIR, pair, 0)
        return carry

    def half_guarded(h, carry):
        @pl.when(h < nstage)
        def _():
            half(h, 0)

        return carry

    lax.fori_loop(0, CHT0 // HALF, half_guarded, 0)
    plsc.subcore_barrier()

    for k in range(ROWS_PER_TILE // CHUNK):
        row0 = rbase + k * CHUNK
        pltpu.sync_copy(agg_s.at[pl.ds(row0, CHUNK)], rp0)
        pltpu.sync_copy(rp0, agg_out.at[c, pl.ds(row0, CHUNK)])


def _sc_edge(y, p, src64, dst2, zy):
    mesh = plsc.VectorSubcoreMesh(core_axis_name="c", subcore_axis_name="s",
                                  num_cores=NC)
    fn = functools.partial(
        pl.kernel,
        mesh=mesh,
        out_type=jax.ShapeDtypeStruct((NC, N_PAD, D_HIDDEN), jnp.float32),
        scratch_types=[
            pltpu.VMEM((HALF, CH2), jnp.int32),
            pltpu.VMEM((HALF, 2 * CH2), jnp.int32),
            pltpu.VMEM((CHUNK, D_HIDDEN), jnp.float32),
            pltpu.VMEM((CHUNK, D_HIDDEN), jnp.float32),
            pltpu.VMEM_SHARED((N_PAD, D_HIDDEN), jnp.float32),
            pltpu.SemaphoreType.DMA,
            pltpu.SemaphoreType.DMA,
            pltpu.SemaphoreType.DMA,
            pltpu.SemaphoreType.DMA,
        ],
    )(_sc_body)
    return fn(y, p, src64, dst2, zy)


def _post_body(agg_ref, b1_ref, w2_ref, b2_ref, batch_ref,
               out_ref, sums_ref, cnts_ref):
    i = pl.program_id(0)

    @pl.when(i == 0)
    def _():
        sums_ref[...] = jnp.zeros_like(sums_ref)
        cnts_ref[...] = jnp.zeros_like(cnts_ref)

    pre = sum(agg_ref[i] for i in range(NC)) + b1_ref[...]
    h = jnp.maximum(pre, 0.0)
    h2 = jnp.dot(h, w2_ref[...],
                 preferred_element_type=jnp.float32,
                 precision=jax.lax.Precision.HIGHEST) + b2_ref[...]

    gid = lax.broadcasted_iota(jnp.int32, (N_GRAPHS, CHUNK), 0)
    mask = (batch_ref[0] == gid).astype(jnp.float32)
    sums_ref[...] = sums_ref[...] + jnp.dot(
        mask, h2, preferred_element_type=jnp.float32,
        precision=jax.lax.Precision.HIGHEST)
    cnts_ref[...] = cnts_ref[...] + jnp.sum(mask, axis=1, keepdims=True)

    @pl.when(i == NBLK - 1)
    def _():
        out_ref[...] = sums_ref[...] / jnp.maximum(cnts_ref[...], 1.0)


def _post(agg2, b1r, w2, b2r, batch3):
    return pl.pallas_call(
        _post_body,
        grid=(NBLK,),
        in_specs=[
            pl.BlockSpec((NC, CHUNK, D_HIDDEN), lambda i: (0, i, 0)),
            pl.BlockSpec((1, D_HIDDEN), lambda i: (0, 0)),
            pl.BlockSpec((D_HIDDEN, D_OUT), lambda i: (0, 0)),
            pl.BlockSpec((1, D_OUT), lambda i: (0, 0)),
            pl.BlockSpec((1, 1, CHUNK), lambda i: (i, 0, 0)),
        ],
        out_specs=pl.BlockSpec((N_GRAPHS, D_OUT), lambda i: (0, 0)),
        out_shape=jax.ShapeDtypeStruct((N_GRAPHS, D_OUT), jnp.float32),
        scratch_shapes=[
            pltpu.VMEM((N_GRAPHS, D_OUT), jnp.float32),
            pltpu.VMEM((N_GRAPHS, 1), jnp.float32),
        ],
    )(agg2, b1r, w2, b2r, batch3)


@jax.jit
def kernel(x, edge_index, edge_attr, batch, W1, b1, W2, b2):
    src = edge_index[0].astype(jnp.int32)
    dst = edge_index[1].astype(jnp.int32)
    pad_e = E_PAD - N_EDGES
    src64 = jnp.concatenate(
        [src, jnp.full((pad_e,), N_NODES, jnp.int32)]).reshape(NCH, CH2)
    dst64 = jnp.concatenate(
        [dst, jnp.full((pad_e,), N_NODES, jnp.int32)]).reshape(NCH, CH2)
    dst2 = jnp.concatenate([dst64, dst64], axis=1)
    x_p = jnp.pad(x, ((0, N_PAD - N_NODES), (0, 0)))
    batch3 = jnp.pad(batch.astype(jnp.int32), (0, N_PAD - N_NODES),
                     constant_values=N_GRAPHS).reshape(NBLK, 1, CHUNK)
    zy = jnp.zeros((CHUNK, D_HIDDEN), jnp.float32)
    w1a = W1[:D_NODE]
    w1b = W1[D_NODE:]
    b1r = b1.reshape(1, D_HIDDEN)
    b2r = b2.reshape(1, D_OUT)

    y = _node_proj(x_p, w1a)
    p = _edge_proj(edge_attr, w1b)
    agg2 = _sc_edge(y, p, src64, dst2, zy)
    return _post(agg2, b1r, W2, b2r, batch3)
